# baseline (speedup 1.0000x reference)
import functools

import jax
import jax.numpy as jnp
from jax import lax
from jax.experimental import pallas as pl
from jax.experimental.pallas import tpu as pltpu
from jax.experimental.pallas import tpu_sc as plsc

BATCH = 16384
DIM = 2048
L = 16
NC = 2
NS = 16
NW = NC * NS
ROWS_PER_W = BATCH // NW
CHUNK_R = 8
NCHUNK = ROWS_PER_W // CHUNK_R
CW = CHUNK_R * DIM


def _sc_body(z_hbm, idx_hbm, out_hbm, idx_v, in_v, out_v):
    wid = lax.axis_index("s") * NC + lax.axis_index("c")
    base = wid * (ROWS_PER_W * DIM)
    pltpu.sync_copy(idx_hbm, idx_v)

    def chunk_body(c, carry):
        off = base + c * CW
        pltpu.sync_copy(z_hbm.at[pl.ds(off, CW)], in_v)

        def col_body(k, carry2):
            colv = idx_v[pl.ds(k * L, L)]
            for r in range(CHUNK_R):
                vals = plsc.load_gather(in_v, [colv + r * DIM])
                out_v[pl.ds(k * L + r * DIM, L)] = vals
            return carry2

        lax.fori_loop(0, DIM // L, col_body, 0, unroll=4)
        pltpu.sync_copy(out_v, out_hbm.at[pl.ds(off, CW)])
        return carry

    lax.fori_loop(0, NCHUNK, chunk_body, 0)


_sc_kernel = functools.partial(
    pl.kernel,
    mesh=plsc.VectorSubcoreMesh(core_axis_name="c", subcore_axis_name="s"),
    out_type=jax.ShapeDtypeStruct((BATCH * DIM,), jnp.float32),
    compiler_params=pltpu.CompilerParams(needs_layout_passes=False),
    scratch_types=[
        pltpu.VMEM((DIM,), jnp.int32),
        pltpu.VMEM((CW,), jnp.float32),
        pltpu.VMEM((CW,), jnp.float32),
    ],
)(_sc_body)


def kernel(z, perm_indices):
    out_flat = _sc_kernel(z.reshape(BATCH * DIM), perm_indices)
    return out_flat.reshape(BATCH, DIM)


# speedup vs baseline: 3.2651x; 3.2651x over previous
import functools

import jax
import jax.numpy as jnp
from jax import lax
from jax.experimental import pallas as pl
from jax.experimental.pallas import tpu as pltpu
from jax.experimental.pallas import tpu_sc as plsc

BATCH = 16384
DIM = 2048
L = 16
NC = 2
NS = 16
NW = NC * NS
ROWS_PER_W = BATCH // NW
CHUNK_R = 8
NCHUNK = ROWS_PER_W // CHUNK_R


def _sc_body(z_hbm, idx_hbm, out_hbm, idx_v, in_v, out_v):
    wid = lax.axis_index("s") * NC + lax.axis_index("c")
    row0 = wid * ROWS_PER_W
    pltpu.sync_copy(idx_hbm, idx_v)

    def chunk_body(c, carry):
        r0 = row0 + c * CHUNK_R
        pltpu.sync_copy(z_hbm.at[pl.ds(r0, CHUNK_R)], in_v)

        @plsc.parallel_loop(0, DIM // L, unroll=4)
        def col_body(k):
            colv = idx_v[pl.ds(k * L, L)]
            for r in range(CHUNK_R):
                rsplat = jnp.full((L,), r, jnp.int32)
                vals = plsc.load_gather(in_v, [rsplat, colv])
                out_v[r, pl.ds(k * L, L)] = vals

        pltpu.sync_copy(out_v, out_hbm.at[pl.ds(r0, CHUNK_R)])
        return carry

    lax.fori_loop(0, NCHUNK, chunk_body, 0)


_sc_kernel = functools.partial(
    pl.kernel,
    mesh=plsc.VectorSubcoreMesh(core_axis_name="c", subcore_axis_name="s"),
    out_type=jax.ShapeDtypeStruct((BATCH, DIM), jnp.float32),
    compiler_params=pltpu.CompilerParams(needs_layout_passes=False),
    scratch_types=[
        pltpu.VMEM((DIM,), jnp.int32),
        pltpu.VMEM((CHUNK_R, DIM), jnp.float32),
        pltpu.VMEM((CHUNK_R, DIM), jnp.float32),
    ],
)(_sc_body)


def kernel(z, perm_indices):
    return _sc_kernel(z, perm_indices)


# speedup vs baseline: 5.8166x; 1.7814x over previous
import functools

import jax
import jax.numpy as jnp
from jax import lax
from jax.experimental import pallas as pl
from jax.experimental.pallas import tpu as pltpu
from jax.experimental.pallas import tpu_sc as plsc

BATCH = 16384
DIM = 2048
L = 16
NC = 2
NS = 16
NW = NC * NS
ROWS_PER_W = BATCH // NW
CHUNK_R = 8
NCHUNK = ROWS_PER_W // CHUNK_R
CHUNK_BYTES = CHUNK_R * DIM * 4


def _sc_body(z_hbm, idx_hbm, out_hbm, idx_v, in_bufs, out_bufs, sems_i, sems_o):
    wid = lax.axis_index("s") * NC + lax.axis_index("c")
    row0 = wid * ROWS_PER_W
    pltpu.sync_copy(idx_hbm, idx_v)

    def in_slice(c):
        return z_hbm.at[pl.ds(row0 + c * CHUNK_R, CHUNK_R)]

    def out_slice(c):
        return out_hbm.at[pl.ds(row0 + c * CHUNK_R, CHUNK_R)]

    def compute(in_v, out_v):
        @plsc.parallel_loop(0, DIM // L, unroll=4)
        def col_body(k):
            colv = idx_v[pl.ds(k * L, L)]
            for r in range(CHUNK_R):
                rsplat = jnp.full((L,), r, jnp.int32)
                vals = plsc.load_gather(in_v, [rsplat, colv])
                out_v[r, pl.ds(k * L, L)] = vals

    for b in range(2):
        pltpu.async_copy(in_slice(b), in_bufs[b], sems_i[b])

    def pair_body(p, carry):
        for b in range(2):
            c = 2 * p + b
            pltpu.make_async_copy(in_slice(0), in_bufs[b], sems_i[b]).wait()

            @pl.when(p > 0)
            def _wait_out():
                pltpu.make_async_copy(out_bufs[b], out_slice(0), sems_o[b]).wait()
            compute(in_bufs[b], out_bufs[b])
            pltpu.async_copy(out_bufs[b], out_slice(c), sems_o[b])
            nxt = jnp.minimum(c + 2, NCHUNK - 1)
            pltpu.async_copy(in_slice(nxt), in_bufs[b], sems_i[b])
        return carry

    lax.fori_loop(0, NCHUNK // 2, pair_body, 0)

    for b in range(2):
        pltpu.make_async_copy(in_slice(0), in_bufs[b], sems_i[b]).wait()
        pltpu.make_async_copy(out_bufs[b], out_slice(0), sems_o[b]).wait()


_sc_kernel = functools.partial(
    pl.kernel,
    mesh=plsc.VectorSubcoreMesh(core_axis_name="c", subcore_axis_name="s"),
    out_type=jax.ShapeDtypeStruct((BATCH, DIM), jnp.float32),
    compiler_params=pltpu.CompilerParams(needs_layout_passes=False),
    scratch_types=[
        pltpu.VMEM((DIM,), jnp.int32),
        [pltpu.VMEM((CHUNK_R, DIM), jnp.float32) for _ in range(2)],
        [pltpu.VMEM((CHUNK_R, DIM), jnp.float32) for _ in range(2)],
        [pltpu.SemaphoreType.DMA for _ in range(2)],
        [pltpu.SemaphoreType.DMA for _ in range(2)],
    ],
)(_sc_body)


def kernel(z, perm_indices):
    return _sc_kernel(z, perm_indices)


# speedup vs baseline: 5.8601x; 1.0075x over previous
import functools

import jax
import jax.numpy as jnp
from jax import lax
from jax.experimental import pallas as pl
from jax.experimental.pallas import tpu as pltpu
from jax.experimental.pallas import tpu_sc as plsc

BATCH = 16384
DIM = 2048
L = 16
NC = 2
NS = 16
NW = NC * NS
ROWS_PER_W = BATCH // NW
CHUNK_R = 8
NCHUNK = ROWS_PER_W // CHUNK_R
CHUNK_BYTES = CHUNK_R * DIM * 4


def _sc_body(z_hbm, idx_hbm, out_hbm, idx_v, in_bufs, out_bufs, sems_i, sems_o):
    wid = lax.axis_index("s") * NC + lax.axis_index("c")
    row0 = wid * ROWS_PER_W
    pltpu.sync_copy(idx_hbm, idx_v)

    def in_slice(c):
        return z_hbm.at[pl.ds(row0 + c * CHUNK_R, CHUNK_R)]

    def out_slice(c):
        return out_hbm.at[pl.ds(row0 + c * CHUNK_R, CHUNK_R)]

    def compute(in_v, out_v):
        @plsc.parallel_loop(0, DIM // L, unroll=4)
        def col_body(k):
            colv = idx_v[pl.ds(k * L, L)]
            for r in range(CHUNK_R):
                rsplat = jnp.full((L,), r, jnp.int32)
                vals = plsc.load_gather(in_v, [rsplat, colv])
                out_v[r, pl.ds(k * L, L)] = vals

    for b in range(2):
        pltpu.async_copy(in_slice(b), in_bufs[b], sems_i[b])

    def pair_body(p, carry):
        for b in range(2):
            c = 2 * p + b
            pltpu.make_async_copy(in_slice(0), in_bufs[b], sems_i[b]).wait()

            @pl.when(p > 0)
            def _wait_out():
                pltpu.make_async_copy(out_bufs[b], out_slice(0), sems_o[b]).wait()
            compute(in_bufs[b], out_bufs[b])
            pltpu.async_copy(out_bufs[b], out_slice(c), sems_o[b])

            @pl.when(c + 2 < NCHUNK)
            def _prefetch():
                pltpu.async_copy(in_slice(c + 2), in_bufs[b], sems_i[b])

        return carry

    lax.fori_loop(0, NCHUNK // 2, pair_body, 0)

    for b in range(2):
        pltpu.make_async_copy(out_bufs[b], out_slice(0), sems_o[b]).wait()


_sc_kernel = functools.partial(
    pl.kernel,
    mesh=plsc.VectorSubcoreMesh(core_axis_name="c", subcore_axis_name="s"),
    out_type=jax.ShapeDtypeStruct((BATCH, DIM), jnp.float32),
    compiler_params=pltpu.CompilerParams(needs_layout_passes=False),
    scratch_types=[
        pltpu.VMEM((DIM,), jnp.int32),
        [pltpu.VMEM((CHUNK_R, DIM), jnp.float32) for _ in range(2)],
        [pltpu.VMEM((CHUNK_R, DIM), jnp.float32) for _ in range(2)],
        [pltpu.SemaphoreType.DMA for _ in range(2)],
        [pltpu.SemaphoreType.DMA for _ in range(2)],
    ],
)(_sc_body)


def kernel(z, perm_indices):
    return _sc_kernel(z, perm_indices)
